# Initial kernel scaffold; baseline (speedup 1.0000x reference)
import jax
import jax.numpy as jnp
from jax.experimental import pallas as pl


def kernel(x, edge_index, edge_attr, batch_ids, We1, be1, We2, be2, Wm1, bm1, Wm2, bm2, Wu1, bu1, Wu2, bu2, Wh1, bh1, Wh2, bh2):
    raise NotImplementedError("write your pallas kernel here")


# speedup vs baseline: 2.0985x; 2.0985x over previous
import functools

import jax
import jax.numpy as jnp
from jax import lax
from jax.experimental import pallas as pl
from jax.experimental.pallas import tpu as pltpu
from jax.experimental.pallas import tpu_sc as plsc

N = 10000
E = 160000
D = 256
DE = 16
H = 256
OUT = 128
DEPTH = 3
G = 64

NC = 2
NS = 16
EPT = E // NS
ROWS_PT = N // NS
K = 100
CH = 100
NSUB = K // CH
NCHUNK = EPT // K

BN = 1000
BE = 2000
F32 = jnp.float32


def _embed_body(x_ref, w1_ref, b1_ref, w2_ref, b2_ref, o_ref):
    h = jnp.maximum(
        jnp.dot(x_ref[...], w1_ref[...], preferred_element_type=F32) + b1_ref[0],
        0.0)
    o_ref[...] = jnp.dot(h, w2_ref[...], preferred_element_type=F32) + b2_ref[0]


def _embed(x, W1, b1, W2, b2):
    return pl.pallas_call(
        _embed_body,
        grid=(N // BN,),
        in_specs=[
            pl.BlockSpec((BN, D), lambda i: (i, 0)),
            pl.BlockSpec((D, H), lambda i: (0, 0)),
            pl.BlockSpec((1, H), lambda i: (0, 0)),
            pl.BlockSpec((H, H), lambda i: (0, 0)),
            pl.BlockSpec((1, H), lambda i: (0, 0)),
        ],
        out_specs=pl.BlockSpec((BN, H), lambda i: (i, 0)),
        out_shape=jax.ShapeDtypeStruct((N, H), F32),
    )(x, W1, b1, W2, b2)


def _q_body(ea_ref, wc_ref, q_ref):
    q_ref[...] = jnp.dot(ea_ref[...], wc_ref[0],
                         preferred_element_type=F32)[None, None]


def _q_tables(edge_attr, Wm1c):
    return pl.pallas_call(
        _q_body,
        grid=(DEPTH, 2, E // BE),
        in_specs=[
            pl.BlockSpec((BE, DE), lambda i, c, e: (e, 0)),
            pl.BlockSpec((1, DE, H // 2), lambda i, c, e: (i, 0, c)),
        ],
        out_specs=pl.BlockSpec((1, 1, BE, H // 2), lambda i, c, e: (i, c, e, 0)),
        out_shape=jax.ShapeDtypeStruct((DEPTH, 2, E, H // 2), F32),
    )(edge_attr, Wm1c)


def _pre_body(x_ref, wa_ref, wb_ref, bm_ref, pa_ref, pb_ref):
    xb = x_ref[...]
    pa_ref[...] = jnp.dot(xb, wa_ref[...], preferred_element_type=F32)
    pb_ref[...] = jnp.dot(xb, wb_ref[...], preferred_element_type=F32) + bm_ref[0]


def _pre(x, Wa, Wb, bm):
    return pl.pallas_call(
        _pre_body,
        grid=(N // BN,),
        in_specs=[
            pl.BlockSpec((BN, H), lambda i: (i, 0)),
            pl.BlockSpec((H, H), lambda i: (0, 0)),
            pl.BlockSpec((H, H), lambda i: (0, 0)),
            pl.BlockSpec((1, H), lambda i: (0, 0)),
        ],
        out_specs=[
            pl.BlockSpec((BN, H), lambda i: (i, 0)),
            pl.BlockSpec((BN, H), lambda i: (i, 0)),
        ],
        out_shape=[
            jax.ShapeDtypeStruct((N, H), F32),
            jax.ShapeDtypeStruct((N, H), F32),
        ],
    )(x, Wa, Wb, bm)


def _upd_body(s_ref, cnt_ref, x_ref, wm2_ref, bm2_ref, wua_ref, wub_ref,
              bu1_ref, wu2_ref, bu2_ref, o_ref):
    s0 = s_ref[0]
    s1 = s_ref[1]
    ssum = (jnp.dot(s0, wm2_ref[0:128, :], preferred_element_type=F32)
            + jnp.dot(s1, wm2_ref[128:256, :], preferred_element_type=F32))
    cnt = cnt_ref[...][:, 0:1]
    aggr = (ssum + cnt * bm2_ref[0]) / jnp.maximum(cnt, 1.0)
    xb = x_ref[...]
    h = jnp.maximum(
        jnp.dot(xb, wua_ref[...], preferred_element_type=F32)
        + jnp.dot(aggr, wub_ref[...], preferred_element_type=F32)
        + bu1_ref[0], 0.0)
    o_ref[...] = jnp.dot(h, wu2_ref[...], preferred_element_type=F32) + bu2_ref[0]


def _update(S, cnt16, x, Wm2i, bm2i, Wua, Wub, bu1i, Wu2i, bu2i):
    return pl.pallas_call(
        _upd_body,
        grid=(N // BN,),
        in_specs=[
            pl.BlockSpec((2, BN, H // 2), lambda i: (0, i, 0)),
            pl.BlockSpec((BN, 16), lambda i: (i, 0)),
            pl.BlockSpec((BN, H), lambda i: (i, 0)),
            pl.BlockSpec((H, H), lambda i: (0, 0)),
            pl.BlockSpec((1, H), lambda i: (0, 0)),
            pl.BlockSpec((H, H), lambda i: (0, 0)),
            pl.BlockSpec((H, H), lambda i: (0, 0)),
            pl.BlockSpec((1, H), lambda i: (0, 0)),
            pl.BlockSpec((H, H), lambda i: (0, 0)),
            pl.BlockSpec((1, H), lambda i: (0, 0)),
        ],
        out_specs=pl.BlockSpec((BN, H), lambda i: (i, 0)),
        out_shape=jax.ShapeDtypeStruct((N, H), F32),
    )(S, cnt16, x, Wm2i, bm2i, Wua, Wub, bu1i, Wu2i, bu2i)


def _pool_body(x_ref, bid_ref, wh1_ref, bh1_ref, wh2_ref, bh2_ref, o_ref,
               acc_ref):
    i = pl.program_id(0)

    @pl.when(i == 0)
    def _init():
        acc_ref[...] = jnp.zeros_like(acc_ref)

    bid = bid_ref[0, 0]
    oh = (lax.broadcasted_iota(jnp.int32, (G, BN), 0)
          == bid[None, :]).astype(F32)
    acc_ref[...] += jnp.dot(oh, x_ref[...], preferred_element_type=F32)

    @pl.when(i == pl.num_programs(0) - 1)
    def _fin():
        h = jnp.maximum(
            jnp.dot(acc_ref[...], wh1_ref[...], preferred_element_type=F32)
            + bh1_ref[0], 0.0)
        o_ref[...] = jnp.dot(h, wh2_ref[...], preferred_element_type=F32) + bh2_ref[0]


def _pool_head(x, bidr, Wh1, bh1, Wh2, bh2):
    return pl.pallas_call(
        _pool_body,
        grid=(N // BN,),
        in_specs=[
            pl.BlockSpec((BN, H), lambda i: (i, 0)),
            pl.BlockSpec((1, 1, BN), lambda i: (i, 0, 0)),
            pl.BlockSpec((H, H), lambda i: (0, 0)),
            pl.BlockSpec((1, H), lambda i: (0, 0)),
            pl.BlockSpec((H, OUT), lambda i: (0, 0)),
            pl.BlockSpec((1, OUT), lambda i: (0, 0)),
        ],
        out_specs=pl.BlockSpec((G, OUT), lambda i: (0, 0)),
        out_shape=jax.ShapeDtypeStruct((G, OUT), F32),
        scratch_shapes=[pltpu.VMEM((G, H), F32)],
    )(x, bidr, Wh1, bh1, Wh2, bh2)


def _make_sc(layer, with_cnt):
    mesh = plsc.VectorSubcoreMesh(core_axis_name="c", subcore_axis_name="s",
                                  num_cores=NC, num_subcores=NS)
    out_type = [jax.ShapeDtypeStruct((2, N, H // 2), F32)]
    if with_cnt:
        out_type.append(jax.ShapeDtypeStruct((N, 16), F32))
    scratch = [
        pltpu.VMEM((NSUB, CH), jnp.int32),
        pltpu.VMEM((NSUB, CH), jnp.int32),
        pltpu.VMEM((NSUB, CH), jnp.int32),
        pltpu.VMEM((K, H // 2), F32),
        pltpu.VMEM((K, H // 2), F32),
        pltpu.VMEM((K, H // 2), F32),
        pltpu.VMEM((CH, 16), F32),
        pltpu.VMEM_SHARED((N, H // 2), F32),
        pltpu.VMEM_SHARED((N, 16), F32),
        pltpu.SemaphoreType.DMA,
        pltpu.SemaphoreType.DMA,
    ]

    def body(pa_hbm, pb_hbm, qall_hbm, gsrc_hbm, gdst_hbm, dstr_hbm, z_hbm,
             zc_hbm, *outs_and_scratch):
        if with_cnt:
            s_out, cnt_out = outs_and_scratch[:2]
            rest = outs_and_scratch[2:]
        else:
            s_out = outs_and_scratch[0]
            cnt_out = None
            rest = outs_and_scratch[1:]
        (isrc, idst, sdst, va, vb, vq, vones, s_sh, c_sh, sem1, sem2) = rest
        cid = lax.axis_index("c")
        sid = lax.axis_index("s")
        myrows = pl.ds(sid * ROWS_PT, ROWS_PT)

        pltpu.sync_copy(z_hbm.at[myrows], s_sh.at[myrows])
        if with_cnt:
            @pl.when(cid == 0)
            def _zc():
                pltpu.sync_copy(zc_hbm.at[myrows], c_sh.at[myrows])

            def _ones_row(r, carry):
                vones[r] = jnp.ones((16,), F32)
                return carry
            lax.fori_loop(0, CH, _ones_row, 0)
        plsc.subcore_barrier()

        def chunk(c, carry):
            base_e = sid * EPT + c * K
            base_r = sid * (EPT // CH) + c * NSUB
            pltpu.sync_copy(gsrc_hbm.at[cid, pl.ds(base_r, NSUB)], isrc)
            pltpu.sync_copy(gdst_hbm.at[cid, pl.ds(base_r, NSUB)], idst)
            pltpu.sync_copy(dstr_hbm.at[pl.ds(base_r, NSUB)], sdst)
            cps = []
            for j in range(NSUB):
                cps.append(pltpu.async_copy(
                    pa_hbm.at[isrc.at[j]], va.at[pl.ds(j * CH, CH)], sem1))
                cps.append(pltpu.async_copy(
                    pb_hbm.at[idst.at[j]], vb.at[pl.ds(j * CH, CH)], sem2))
            pltpu.sync_copy(qall_hbm.at[layer, cid, pl.ds(base_e, K)], vq)
            for cp in cps:
                cp.wait()

            def rowf(r, rc):
                for j in range(H // 2 // 16):
                    sl = pl.ds(j * 16, 16)
                    va[r, sl] = jnp.maximum(va[r, sl] + vb[r, sl] + vq[r, sl],
                                            0.0)
                return rc
            lax.fori_loop(0, K, rowf, 0)

            for j in range(NSUB):
                pltpu.sync_copy(va.at[pl.ds(j * CH, CH)], s_sh.at[sdst.at[j]],
                                add=True)
            if with_cnt:
                @pl.when(cid == 0)
                def _cnt():
                    for j in range(NSUB):
                        pltpu.sync_copy(vones, c_sh.at[sdst.at[j]], add=True)
            return carry
        lax.fori_loop(0, NCHUNK, chunk, 0)
        plsc.subcore_barrier()

        pltpu.sync_copy(s_sh.at[myrows], s_out.at[cid, myrows])
        if with_cnt:
            @pl.when(cid == 0)
            def _co():
                pltpu.sync_copy(c_sh.at[myrows], cnt_out.at[myrows])

    return pl.kernel(body, out_type=tuple(out_type), mesh=mesh,
                     scratch_types=scratch,
                     compiler_params=pltpu.CompilerParams(
                         use_tc_tiling_on_sc=False))


_sc_first = _make_sc(0, True)
_sc_rest = [_make_sc(i, False) for i in range(1, DEPTH)]


def kernel(x, edge_index, edge_attr, batch_ids, We1, be1, We2, be2,
           Wm1, bm1, Wm2, bm2, Wu1, bu1, Wu2, bu2, Wh1, bh1, Wh2, bh2):
    src = edge_index[0].astype(jnp.int32)
    dst = edge_index[1].astype(jnp.int32)
    gsrc = jnp.stack([2 * src, 2 * src + 1]).reshape(2, E // CH, CH)
    gdst = jnp.stack([2 * dst, 2 * dst + 1]).reshape(2, E // CH, CH)
    dstr = dst.reshape(E // CH, CH)
    zrow = jnp.zeros((N, H // 2), F32)
    zc = jnp.zeros((N, 16), F32)
    bidr = batch_ids.astype(jnp.int32).reshape(N // BN, 1, BN)

    h = _embed(x, We1, be1.reshape(1, H), We2, be2.reshape(1, H))
    qall = _q_tables(edge_attr, Wm1[:, 2 * H:, :])

    cnt16 = None
    for i in range(DEPTH):
        pa, pb = _pre(h, Wm1[i, :H, :], Wm1[i, H:2 * H, :],
                      bm1[i].reshape(1, H))
        pa2 = pa.reshape(2 * N, H // 2)
        pb2 = pb.reshape(2 * N, H // 2)
        if i == 0:
            S, cnt16 = _sc_first(pa2, pb2, qall, gsrc, gdst, dstr, zrow, zc)
        else:
            (S,) = _sc_rest[i - 1](pa2, pb2, qall, gsrc, gdst, dstr, zrow, zc)
        h = _update(S, cnt16, h, Wm2[i], bm2[i].reshape(1, H),
                    Wu1[i, :H, :], Wu1[i, H:, :], bu1[i].reshape(1, H),
                    Wu2[i], bu2[i].reshape(1, H))

    return _pool_head(h, bidr, Wh1, bh1.reshape(1, H), Wh2, bh2.reshape(1, OUT))
